# baseline (speedup 1.0000x reference)
import functools

import jax
import jax.numpy as jnp
from jax import lax
from jax.experimental import pallas as pl
from jax.experimental.pallas import tpu as pltpu
from jax.experimental.pallas import tpu_sc as plsc

NC = 2
NS = 16
NW = NC * NS
LANES = 16
CHUNK = 128


def _make_deg_kernel(npad, e_t):
    mesh = plsc.VectorSubcoreMesh(core_axis_name="c", subcore_axis_name="s")

    @functools.partial(
        pl.kernel,
        out_type=jax.ShapeDtypeStruct((NW, npad), jnp.float32),
        mesh=mesh,
        scratch_types=[
            pltpu.VMEM((e_t,), jnp.int32),
            pltpu.VMEM((npad,), jnp.float32),
        ],
        compiler_params=pltpu.CompilerParams(needs_layout_passes=False),
    )
    def deg_kernel(dst_hbm, out_hbm, dst_v, deg_v):
        c = lax.axis_index("c")
        s = lax.axis_index("s")
        wid = s * NC + c
        zeros16 = jnp.zeros((LANES,), jnp.float32)

        def zero_body(i, _):
            deg_v[pl.ds(i * LANES, LANES)] = zeros16
            return 0

        lax.fori_loop(0, npad // LANES, zero_body, 0)
        pltpu.sync_copy(dst_hbm.at[pl.ds(wid * e_t, e_t)], dst_v)
        ones16 = jnp.ones((LANES,), jnp.float32)

        def body(i, _):
            idx = dst_v[pl.ds(i * LANES, LANES)]
            plsc.addupdate_scatter(deg_v, [idx], ones16)
            return 0

        lax.fori_loop(0, e_t // LANES, body, 0)
        pltpu.sync_copy(deg_v, out_hbm.at[wid])

    return deg_kernel


def _make_agg_kernel(nacc, d, n_chunks):
    mesh = plsc.VectorSubcoreMesh(core_axis_name="c", subcore_axis_name="s")
    rows_per_tile = nacc // NS

    @functools.partial(
        pl.kernel,
        out_type=jax.ShapeDtypeStruct((NC, nacc, d), jnp.float32),
        mesh=mesh,
        scratch_types=[
            pltpu.VMEM((n_chunks, CHUNK), jnp.int32),
            pltpu.VMEM((2, CHUNK), jnp.int32),
            pltpu.VMEM((CHUNK, d), jnp.float32),
            pltpu.VMEM((CHUNK, d), jnp.float32),
            pltpu.VMEM_SHARED((nacc, d), jnp.float32),
            pltpu.SemaphoreType.DMA,
            pltpu.SemaphoreType.DMA,
            pltpu.SemaphoreType.DMA,
            pltpu.SemaphoreType.DMA,
        ],
        compiler_params=pltpu.CompilerParams(needs_layout_passes=False),
    )
    def agg_kernel(src_hbm, dst_hbm, g_hbm, out_hbm, src_v, dst_d, rows_v,
                   rows_w, acc_sh, sem, sem2, sem3, sem4):
        c = lax.axis_index("c")
        s = lax.axis_index("s")
        wid = s * NC + c
        zeros16 = jnp.zeros((LANES,), jnp.float32)

        def zero_rows(i, _):
            for j in range(d // LANES):
                rows_v[i, pl.ds(j * LANES, LANES)] = zeros16
            return 0

        lax.fori_loop(0, CHUNK, zero_rows, 0)
        base = s * rows_per_tile

        def zero_acc(k, _):
            pltpu.sync_copy(rows_v, acc_sh.at[pl.ds(base + k * CHUNK, CHUNK)])
            return 0

        lax.fori_loop(0, rows_per_tile // CHUNK, zero_acc, 0)
        rem = rows_per_tile % CHUNK
        if rem:
            pltpu.sync_copy(
                rows_v.at[pl.ds(0, rem)],
                acc_sh.at[pl.ds(base + (rows_per_tile // CHUNK) * CHUNK, rem)],
            )
        plsc.subcore_barrier()

        pltpu.sync_copy(src_hbm.at[wid], src_v)

        n_pairs = n_chunks // 2
        pltpu.async_copy(g_hbm.at[src_v.at[0]], rows_v, sem)
        pltpu.async_copy(dst_hbm.at[wid, 0], dst_d.at[0], sem3)

        def pair_body(jj, _):
            j = 2 * jj
            pltpu.async_copy(g_hbm.at[src_v.at[j + 1]], rows_w, sem2)
            pltpu.async_copy(dst_hbm.at[wid, j + 1], dst_d.at[1], sem4)
            pltpu.make_async_copy(g_hbm.at[src_v.at[j]], rows_v, sem).wait()
            pltpu.make_async_copy(dst_hbm.at[wid, j], dst_d.at[0],
                                  sem3).wait()
            pltpu.sync_copy(rows_v, acc_sh.at[dst_d.at[0]], add=True)

            @pl.when(jj + 1 < n_pairs)
            def _():
                pltpu.async_copy(g_hbm.at[src_v.at[j + 2]], rows_v, sem)
                pltpu.async_copy(dst_hbm.at[wid, j + 2], dst_d.at[0], sem3)

            pltpu.make_async_copy(g_hbm.at[src_v.at[j + 1]], rows_w,
                                  sem2).wait()
            pltpu.make_async_copy(dst_hbm.at[wid, j + 1], dst_d.at[1],
                                  sem4).wait()
            pltpu.sync_copy(rows_w, acc_sh.at[dst_d.at[1]], add=True)
            return 0

        lax.fori_loop(0, n_pairs, pair_body, 0)
        plsc.subcore_barrier()
        pltpu.sync_copy(acc_sh.at[pl.ds(base, rows_per_tile)],
                        out_hbm.at[c, pl.ds(base, rows_per_tile)])

    return agg_kernel


def _g_body(x_ref, w_ref, degp_ref, g_ref):
    deg = jnp.sum(degp_ref[...], axis=1, keepdims=True) + 1.0
    dinv = lax.rsqrt(deg)
    h = jnp.dot(x_ref[...], w_ref[...], preferred_element_type=jnp.float32)
    g_ref[...] = h * dinv


def _out_body(acc_ref, g_ref, degp_ref, b_ref, o_ref):
    deg = jnp.sum(degp_ref[...], axis=1, keepdims=True) + 1.0
    dinv = lax.rsqrt(deg)
    a = acc_ref[0] + acc_ref[1]
    o_ref[...] = dinv * (a + g_ref[...]) + b_ref[...]


def kernel(x, edge_index, W, b):
    n, d_in = x.shape
    d_out = W.shape[1]
    e = edge_index.shape[1]

    src = edge_index[0].astype(jnp.int32)
    dst = edge_index[1].astype(jnp.int32)

    npad = ((n + LANES - 1) // LANES) * LANES
    rpt = (((n + 1 + NS - 1) // NS + 7) // 8) * 8
    nacc = rpt * NS
    e_t = e // NW
    n_chunks = (e + NW * CHUNK - 1) // (NW * CHUNK)
    n_chunks += n_chunks % 2
    ept = n_chunks * CHUNK
    n_pad_edges = ept * NW - e

    srcp = jnp.concatenate(
        [src, jnp.zeros((n_pad_edges,), jnp.int32)]).reshape(NW, n_chunks, CHUNK)
    dstp = jnp.concatenate(
        [dst, jnp.full((n_pad_edges,), n, jnp.int32)]).reshape(NW, n_chunks, CHUNK)

    degp = _make_deg_kernel(npad, e_t)(dst)
    degp_t = jnp.transpose(degp)[:n]

    bn = 2000
    grid = (n // bn,)
    g = pl.pallas_call(
        _g_body,
        grid=grid,
        in_specs=[
            pl.BlockSpec((bn, d_in), lambda i: (i, 0)),
            pl.BlockSpec((d_in, d_out), lambda i: (0, 0)),
            pl.BlockSpec((bn, NW), lambda i: (i, 0)),
        ],
        out_specs=pl.BlockSpec((bn, d_out), lambda i: (i, 0)),
        out_shape=jax.ShapeDtypeStruct((n, d_out), jnp.float32),
    )(x, W, degp_t)

    acc = _make_agg_kernel(nacc, d_out, n_chunks)(srcp, dstp, g)

    out = pl.pallas_call(
        _out_body,
        grid=grid,
        in_specs=[
            pl.BlockSpec((NC, bn, d_out), lambda i: (0, i, 0)),
            pl.BlockSpec((bn, d_out), lambda i: (i, 0)),
            pl.BlockSpec((bn, NW), lambda i: (i, 0)),
            pl.BlockSpec((1, d_out), lambda i: (0, 0)),
        ],
        out_specs=pl.BlockSpec((bn, d_out), lambda i: (i, 0)),
        out_shape=jax.ShapeDtypeStruct((n, d_out), jnp.float32),
    )(acc, g, degp_t, b.reshape(1, d_out))
    return out


# speedup vs baseline: 3.0520x; 3.0520x over previous
import functools

import jax
import jax.numpy as jnp
from jax import lax
from jax.experimental import pallas as pl
from jax.experimental.pallas import tpu as pltpu
from jax.experimental.pallas import tpu_sc as plsc

NC = 2
NS = 16
NW = NC * NS
LANES = 16
CHUNK = 128


def _make_deg_kernel(npad, e_t):
    mesh = plsc.VectorSubcoreMesh(core_axis_name="c", subcore_axis_name="s")

    @functools.partial(
        pl.kernel,
        out_type=jax.ShapeDtypeStruct((NW, npad), jnp.float32),
        mesh=mesh,
        scratch_types=[
            pltpu.VMEM((e_t,), jnp.int32),
            pltpu.VMEM((npad,), jnp.float32),
        ],
        compiler_params=pltpu.CompilerParams(needs_layout_passes=False),
    )
    def deg_kernel(dst_hbm, out_hbm, dst_v, deg_v):
        c = lax.axis_index("c")
        s = lax.axis_index("s")
        wid = s * NC + c
        zeros16 = jnp.zeros((LANES,), jnp.float32)

        def zero_body(i, _):
            deg_v[pl.ds(i * LANES, LANES)] = zeros16
            return 0

        lax.fori_loop(0, npad // LANES, zero_body, 0)
        pltpu.sync_copy(dst_hbm.at[pl.ds(wid * e_t, e_t)], dst_v)
        ones16 = jnp.ones((LANES,), jnp.float32)

        def body(i, _):
            idx = dst_v[pl.ds(i * LANES, LANES)]
            plsc.addupdate_scatter(deg_v, [idx], ones16)
            return 0

        lax.fori_loop(0, e_t // LANES, body, 0)
        pltpu.sync_copy(deg_v, out_hbm.at[wid])

    return deg_kernel


def _make_agg_kernel(nacc, d, n_chunks):
    mesh = plsc.VectorSubcoreMesh(core_axis_name="c", subcore_axis_name="s")
    rows_per_tile = nacc // NS

    @functools.partial(
        pl.kernel,
        out_type=jax.ShapeDtypeStruct((NC, nacc, d), jnp.float32),
        mesh=mesh,
        scratch_types=[
            pltpu.VMEM((n_chunks, CHUNK), jnp.int32),
            pltpu.VMEM((2, CHUNK), jnp.int32),
            pltpu.VMEM((CHUNK, d), jnp.float32),
            pltpu.VMEM((CHUNK, d), jnp.float32),
            pltpu.VMEM_SHARED((nacc, d), jnp.float32),
            pltpu.SemaphoreType.DMA,
            pltpu.SemaphoreType.DMA,
            pltpu.SemaphoreType.DMA,
            pltpu.SemaphoreType.DMA,
        ],
        compiler_params=pltpu.CompilerParams(needs_layout_passes=False),
    )
    def agg_kernel(src_hbm, dst_hbm, g_hbm, out_hbm, src_v, dst_d, rows_v,
                   rows_w, acc_sh, sem, sem2, sem3, sem4):
        c = lax.axis_index("c")
        s = lax.axis_index("s")
        wid = s * NC + c
        zeros16 = jnp.zeros((LANES,), jnp.float32)

        def zero_rows(i, _):
            for j in range(d // LANES):
                rows_v[i, pl.ds(j * LANES, LANES)] = zeros16
            return 0

        lax.fori_loop(0, CHUNK, zero_rows, 0)
        base = s * rows_per_tile

        def zero_acc(k, _):
            pltpu.sync_copy(rows_v, acc_sh.at[pl.ds(base + k * CHUNK, CHUNK)])
            return 0

        lax.fori_loop(0, rows_per_tile // CHUNK, zero_acc, 0)
        rem = rows_per_tile % CHUNK
        if rem:
            pltpu.sync_copy(
                rows_v.at[pl.ds(0, rem)],
                acc_sh.at[pl.ds(base + (rows_per_tile // CHUNK) * CHUNK, rem)],
            )
        plsc.subcore_barrier()

        pltpu.sync_copy(src_hbm.at[wid], src_v)

        n_pairs = n_chunks // 2
        pltpu.async_copy(g_hbm.at[src_v.at[0]], rows_v, sem)
        pltpu.async_copy(dst_hbm.at[wid, 0], dst_d.at[0], sem3)

        def pair_body(jj, _):
            j = 2 * jj
            pltpu.async_copy(g_hbm.at[src_v.at[j + 1]], rows_w, sem2)
            pltpu.async_copy(dst_hbm.at[wid, j + 1], dst_d.at[1], sem4)
            pltpu.make_async_copy(g_hbm.at[src_v.at[j]], rows_v, sem).wait()
            pltpu.make_async_copy(dst_hbm.at[wid, j], dst_d.at[0],
                                  sem3).wait()
            pltpu.sync_copy(rows_v, acc_sh.at[dst_d.at[0]], add=True)

            @pl.when(jj + 1 < n_pairs)
            def _():
                pltpu.async_copy(g_hbm.at[src_v.at[j + 2]], rows_v, sem)
                pltpu.async_copy(dst_hbm.at[wid, j + 2], dst_d.at[0], sem3)

            pltpu.make_async_copy(g_hbm.at[src_v.at[j + 1]], rows_w,
                                  sem2).wait()
            pltpu.make_async_copy(dst_hbm.at[wid, j + 1], dst_d.at[1],
                                  sem4).wait()
            pltpu.sync_copy(rows_w, acc_sh.at[dst_d.at[1]], add=True)
            return 0

        lax.fori_loop(0, n_pairs, pair_body, 0)
        plsc.subcore_barrier()
        pltpu.sync_copy(acc_sh.at[pl.ds(base, rows_per_tile)],
                        out_hbm.at[c, pl.ds(base, rows_per_tile)])

    return agg_kernel


def _g_body(x_ref, w_ref, degp_ref, g_ref):
    deg = jnp.sum(degp_ref[...], axis=1, keepdims=True) + 1.0
    dinv = lax.rsqrt(deg)
    h = jnp.dot(x_ref[...], w_ref[...], preferred_element_type=jnp.float32)
    g_ref[...] = h * dinv


def _out_body(acc_ref, g_ref, degp_ref, b_ref, o_ref):
    deg = jnp.sum(degp_ref[...], axis=1, keepdims=True) + 1.0
    dinv = lax.rsqrt(deg)
    a = acc_ref[0] + acc_ref[1]
    o_ref[...] = dinv * (a + g_ref[...]) + b_ref[...]


def kernel(x, edge_index, W, b):
    n, d_in = x.shape
    d_out = W.shape[1]
    e = edge_index.shape[1]

    src = edge_index[0].astype(jnp.int32)
    dst = edge_index[1].astype(jnp.int32)

    npad = ((n + LANES - 1) // LANES) * LANES
    rpt = (((n + 1 + NS - 1) // NS + 7) // 8) * 8
    nacc = rpt * NS
    e_t = e // NW
    n_chunks = (e + NW * CHUNK - 1) // (NW * CHUNK)
    n_chunks += n_chunks % 2
    ept = n_chunks * CHUNK
    n_pad_edges = ept * NW - e

    pad_iota = lax.iota(jnp.int32, n_pad_edges)
    srcp = jnp.concatenate(
        [src, pad_iota % n]).reshape(NW, n_chunks, CHUNK)
    dstp = jnp.concatenate(
        [dst, n + pad_iota % (nacc - n)]).reshape(NW, n_chunks, CHUNK)

    degp = _make_deg_kernel(npad, e_t)(dst)
    degp_t = jnp.transpose(degp)[:n]

    bn = 2000
    grid = (n // bn,)
    g = pl.pallas_call(
        _g_body,
        grid=grid,
        in_specs=[
            pl.BlockSpec((bn, d_in), lambda i: (i, 0)),
            pl.BlockSpec((d_in, d_out), lambda i: (0, 0)),
            pl.BlockSpec((bn, NW), lambda i: (i, 0)),
        ],
        out_specs=pl.BlockSpec((bn, d_out), lambda i: (i, 0)),
        out_shape=jax.ShapeDtypeStruct((n, d_out), jnp.float32),
    )(x, W, degp_t)

    acc = _make_agg_kernel(nacc, d_out, n_chunks)(srcp, dstp, g)

    out = pl.pallas_call(
        _out_body,
        grid=grid,
        in_specs=[
            pl.BlockSpec((NC, bn, d_out), lambda i: (0, i, 0)),
            pl.BlockSpec((bn, d_out), lambda i: (i, 0)),
            pl.BlockSpec((bn, NW), lambda i: (i, 0)),
            pl.BlockSpec((1, d_out), lambda i: (0, 0)),
        ],
        out_specs=pl.BlockSpec((bn, d_out), lambda i: (i, 0)),
        out_shape=jax.ShapeDtypeStruct((n, d_out), jnp.float32),
    )(acc, g, degp_t, b.reshape(1, d_out))
    return out


# speedup vs baseline: 3.2095x; 1.0516x over previous
import functools

import jax
import jax.numpy as jnp
from jax import lax
from jax.experimental import pallas as pl
from jax.experimental.pallas import tpu as pltpu
from jax.experimental.pallas import tpu_sc as plsc

NC = 2
NS = 16
NW = NC * NS
LANES = 16
CHUNK = 128


def _make_deg_kernel(nacc, n_chunks):
    mesh = plsc.VectorSubcoreMesh(core_axis_name="c", subcore_axis_name="s")

    @functools.partial(
        pl.kernel,
        out_type=jax.ShapeDtypeStruct((NW, nacc), jnp.float32),
        mesh=mesh,
        scratch_types=[
            pltpu.VMEM((n_chunks, CHUNK), jnp.int32),
            pltpu.VMEM((nacc,), jnp.float32),
        ],
        compiler_params=pltpu.CompilerParams(needs_layout_passes=False),
    )
    def deg_kernel(edges_hbm, out_hbm, dst_v, deg_v):
        c = lax.axis_index("c")
        s = lax.axis_index("s")
        wid = s * NC + c
        zeros16 = jnp.zeros((LANES,), jnp.float32)

        def zero_body(i, _):
            deg_v[pl.ds(i * LANES, LANES)] = zeros16
            return 0

        lax.fori_loop(0, nacc // LANES, zero_body, 0)
        pltpu.sync_copy(edges_hbm.at[1, wid], dst_v)
        ones16 = jnp.ones((LANES,), jnp.float32)

        def body(r, _):
            for k in range(CHUNK // LANES):
                idx = dst_v[r, pl.ds(k * LANES, LANES)]
                plsc.addupdate_scatter(deg_v, [idx], ones16)
            return 0

        lax.fori_loop(0, n_chunks, body, 0)
        pltpu.sync_copy(deg_v, out_hbm.at[wid])

    return deg_kernel


def _make_agg_kernel(nacc, d, n_chunks):
    mesh = plsc.VectorSubcoreMesh(core_axis_name="c", subcore_axis_name="s")
    rows_per_tile = nacc // NS

    @functools.partial(
        pl.kernel,
        out_type=jax.ShapeDtypeStruct((NC, nacc, d), jnp.float32),
        mesh=mesh,
        scratch_types=[
            pltpu.VMEM((n_chunks, CHUNK), jnp.int32),
            pltpu.VMEM((2, CHUNK), jnp.int32),
            pltpu.VMEM((CHUNK, d), jnp.float32),
            pltpu.VMEM((CHUNK, d), jnp.float32),
            pltpu.VMEM_SHARED((nacc, d), jnp.float32),
            pltpu.SemaphoreType.DMA,
            pltpu.SemaphoreType.DMA,
            pltpu.SemaphoreType.DMA,
            pltpu.SemaphoreType.DMA,
        ],
        compiler_params=pltpu.CompilerParams(needs_layout_passes=False),
    )
    def agg_kernel(edges_hbm, g_hbm, out_hbm, src_v, dst_d, rows_v,
                   rows_w, acc_sh, sem, sem2, sem3, sem4):
        c = lax.axis_index("c")
        s = lax.axis_index("s")
        wid = s * NC + c
        zeros16 = jnp.zeros((LANES,), jnp.float32)

        def zero_rows(i, _):
            for j in range(d // LANES):
                rows_v[i, pl.ds(j * LANES, LANES)] = zeros16
            return 0

        lax.fori_loop(0, CHUNK, zero_rows, 0)
        base = s * rows_per_tile

        def zero_acc(k, _):
            pltpu.sync_copy(rows_v, acc_sh.at[pl.ds(base + k * CHUNK, CHUNK)])
            return 0

        lax.fori_loop(0, rows_per_tile // CHUNK, zero_acc, 0)
        rem = rows_per_tile % CHUNK
        if rem:
            pltpu.sync_copy(
                rows_v.at[pl.ds(0, rem)],
                acc_sh.at[pl.ds(base + (rows_per_tile // CHUNK) * CHUNK, rem)],
            )
        plsc.subcore_barrier()

        pltpu.sync_copy(edges_hbm.at[0, wid], src_v)

        n_pairs = n_chunks // 2
        pltpu.async_copy(g_hbm.at[src_v.at[0]], rows_v, sem)
        pltpu.async_copy(edges_hbm.at[1, wid, 0], dst_d.at[0], sem3)

        def pair_body(jj, _):
            j = 2 * jj
            pltpu.async_copy(g_hbm.at[src_v.at[j + 1]], rows_w, sem2)
            pltpu.async_copy(edges_hbm.at[1, wid, j + 1], dst_d.at[1], sem4)
            pltpu.make_async_copy(g_hbm.at[src_v.at[j]], rows_v, sem).wait()
            pltpu.make_async_copy(edges_hbm.at[1, wid, j], dst_d.at[0],
                                  sem3).wait()
            pltpu.sync_copy(rows_v, acc_sh.at[dst_d.at[0]], add=True)

            @pl.when(jj + 1 < n_pairs)
            def _():
                pltpu.async_copy(g_hbm.at[src_v.at[j + 2]], rows_v, sem)
                pltpu.async_copy(edges_hbm.at[1, wid, j + 2], dst_d.at[0],
                                 sem3)

            pltpu.make_async_copy(g_hbm.at[src_v.at[j + 1]], rows_w,
                                  sem2).wait()
            pltpu.make_async_copy(edges_hbm.at[1, wid, j + 1], dst_d.at[1],
                                  sem4).wait()
            pltpu.sync_copy(rows_w, acc_sh.at[dst_d.at[1]], add=True)
            return 0

        lax.fori_loop(0, n_pairs, pair_body, 0)
        plsc.subcore_barrier()
        pltpu.sync_copy(acc_sh.at[pl.ds(base, rows_per_tile)],
                        out_hbm.at[c, pl.ds(base, rows_per_tile)])

    return agg_kernel


def _dinv_col(degp_block):
    deg_row = jnp.sum(degp_block, axis=0, keepdims=True) + 1.0
    return lax.rsqrt(jnp.transpose(deg_row))


def _g_body(x_ref, w_ref, degp_ref, g_ref):
    h = jnp.dot(x_ref[...], w_ref[...], preferred_element_type=jnp.float32)
    g_ref[...] = h * _dinv_col(degp_ref[...])


def _out_body(acc_ref, g_ref, degp_ref, b_ref, o_ref):
    a = acc_ref[0] + acc_ref[1]
    o_ref[...] = _dinv_col(degp_ref[...]) * (a + g_ref[...]) + b_ref[...]


def kernel(x, edge_index, W, b):
    n, d_in = x.shape
    d_out = W.shape[1]
    e = edge_index.shape[1]

    rpt = (((n + 1 + NS - 1) // NS + 7) // 8) * 8
    nacc = rpt * NS
    n_chunks = (e + NW * CHUNK - 1) // (NW * CHUNK)
    n_chunks += n_chunks % 2
    ept = n_chunks * CHUNK
    n_pad_edges = ept * NW - e

    pad_iota = lax.iota(jnp.int32, n_pad_edges)
    pads = jnp.stack([pad_iota % n, n + pad_iota % (nacc - n)])
    edges = jnp.concatenate(
        [edge_index.astype(jnp.int32), pads], axis=1
    ).reshape(2, NW, n_chunks, CHUNK)

    degp = _make_deg_kernel(nacc, n_chunks)(edges)

    bn = 1024
    grid = ((n + bn - 1) // bn,)
    g = pl.pallas_call(
        _g_body,
        grid=grid,
        in_specs=[
            pl.BlockSpec((bn, d_in), lambda i: (i, 0)),
            pl.BlockSpec((d_in, d_out), lambda i: (0, 0)),
            pl.BlockSpec((NW, bn), lambda i: (0, i)),
        ],
        out_specs=pl.BlockSpec((bn, d_out), lambda i: (i, 0)),
        out_shape=jax.ShapeDtypeStruct((n, d_out), jnp.float32),
    )(x, W, degp)

    acc = _make_agg_kernel(nacc, d_out, n_chunks)(edges, g)

    out = pl.pallas_call(
        _out_body,
        grid=grid,
        in_specs=[
            pl.BlockSpec((NC, bn, d_out), lambda i: (0, i, 0)),
            pl.BlockSpec((bn, d_out), lambda i: (i, 0)),
            pl.BlockSpec((NW, bn), lambda i: (0, i)),
            pl.BlockSpec((1, d_out), lambda i: (0, 0)),
        ],
        out_specs=pl.BlockSpec((bn, d_out), lambda i: (i, 0)),
        out_shape=jax.ShapeDtypeStruct((n, d_out), jnp.float32),
    )(acc, g, degp, b.reshape(1, d_out))
    return out


# speedup vs baseline: 3.3321x; 1.0382x over previous
import functools

import jax
import jax.numpy as jnp
from jax import lax
from jax.experimental import pallas as pl
from jax.experimental.pallas import tpu as pltpu
from jax.experimental.pallas import tpu_sc as plsc

NC = 2
NS = 16
NW = NC * NS
LANES = 16
CHUNK = 128


def _make_deg_kernel(nacc, n_chunks, bw, off_e):
    mesh = plsc.VectorSubcoreMesh(core_axis_name="c", subcore_axis_name="s")
    ept = n_chunks * CHUNK

    @functools.partial(
        pl.kernel,
        out_type=jax.ShapeDtypeStruct((NW, nacc), jnp.float32),
        mesh=mesh,
        scratch_types=[
            pltpu.VMEM((ept,), jnp.int32),
            pltpu.VMEM((nacc,), jnp.float32),
        ],
        compiler_params=pltpu.CompilerParams(needs_layout_passes=False),
    )
    def deg_kernel(edges_hbm, out_hbm, dst_v, deg_v):
        c = lax.axis_index("c")
        s = lax.axis_index("s")
        wid = s * NC + c
        zeros16 = jnp.zeros((LANES,), jnp.float32)

        def zero_body(i, _):
            deg_v[pl.ds(i * LANES, LANES)] = zeros16
            return 0

        lax.fori_loop(0, nacc // LANES, zero_body, 0)

        @pl.when(wid < bw)
        def _():
            pltpu.sync_copy(edges_hbm.at[1, pl.ds(wid * ept, ept)], dst_v)

        if off_e:
            @pl.when(wid == bw)
            def _():
                pltpu.sync_copy(edges_hbm.at[1, pl.ds(bw * ept, off_e)],
                                dst_v.at[pl.ds(0, off_e)])

        n_real = jnp.where(wid < bw, ept, jnp.where(wid == bw, off_e, 0))
        ones16 = jnp.ones((LANES,), jnp.float32)

        def body(i, _):
            idx = dst_v[pl.ds(i * LANES, LANES)]
            plsc.addupdate_scatter(deg_v, [idx], ones16)
            return 0

        lax.fori_loop(0, n_real // LANES, body, 0)
        pltpu.sync_copy(deg_v, out_hbm.at[wid])

    return deg_kernel


def _make_agg_kernel(nacc, d, n_chunks, bw, off_e):
    mesh = plsc.VectorSubcoreMesh(core_axis_name="c", subcore_axis_name="s")
    rows_per_tile = nacc // NS
    ept = n_chunks * CHUNK

    @functools.partial(
        pl.kernel,
        out_type=jax.ShapeDtypeStruct((NC, nacc, d), jnp.float32),
        mesh=mesh,
        scratch_types=[
            pltpu.VMEM((ept,), jnp.int32),
            pltpu.VMEM((2, CHUNK), jnp.int32),
            pltpu.VMEM((CHUNK, d), jnp.float32),
            pltpu.VMEM((CHUNK, d), jnp.float32),
            pltpu.VMEM_SHARED((nacc, d), jnp.float32),
            pltpu.SemaphoreType.DMA,
            pltpu.SemaphoreType.DMA,
            pltpu.SemaphoreType.DMA,
            pltpu.SemaphoreType.DMA,
        ],
        compiler_params=pltpu.CompilerParams(needs_layout_passes=False),
    )
    def agg_kernel(edges_hbm, g_hbm, out_hbm, src_v, dst_d, rows_v,
                   rows_w, acc_sh, sem, sem2, sem3, sem4):
        c = lax.axis_index("c")
        s = lax.axis_index("s")
        wid = s * NC + c
        ebase = wid * ept
        zeros16 = jnp.zeros((LANES,), jnp.float32)

        def zero_rows(i, _):
            for j in range(d // LANES):
                rows_v[i, pl.ds(j * LANES, LANES)] = zeros16
            return 0

        lax.fori_loop(0, CHUNK, zero_rows, 0)
        base = s * rows_per_tile

        def zero_acc(k, _):
            pltpu.sync_copy(rows_v, acc_sh.at[pl.ds(base + k * CHUNK, CHUNK)])
            return 0

        lax.fori_loop(0, rows_per_tile // CHUNK, zero_acc, 0)
        rem = rows_per_tile % CHUNK
        if rem:
            pltpu.sync_copy(
                rows_v.at[pl.ds(0, rem)],
                acc_sh.at[pl.ds(base + (rows_per_tile // CHUNK) * CHUNK, rem)],
            )
        plsc.subcore_barrier()

        @pl.when(wid < bw)
        def _():
            pltpu.sync_copy(edges_hbm.at[0, pl.ds(ebase, ept)], src_v)

        if off_e:
            @pl.when(wid == bw)
            def _():
                pltpu.sync_copy(edges_hbm.at[0, pl.ds(bw * ept, off_e)],
                                src_v.at[pl.ds(0, off_e)])

        n_real = jnp.where(wid < bw, ept, jnp.where(wid == bw, off_e, 0))
        n_pairs = n_real // (2 * CHUNK)

        def src_at(j):
            return src_v.at[pl.ds(j * CHUNK, CHUNK)]

        def dst_hbm_at(j):
            return edges_hbm.at[1, pl.ds(ebase + j * CHUNK, CHUNK)]

        @pl.when(n_pairs > 0)
        def _():
            pltpu.async_copy(g_hbm.at[src_at(0)], rows_v, sem)
            pltpu.async_copy(dst_hbm_at(0), dst_d.at[0], sem3)

        def pair_body(jj, _):
            j = 2 * jj
            pltpu.async_copy(g_hbm.at[src_at(j + 1)], rows_w, sem2)
            pltpu.async_copy(dst_hbm_at(j + 1), dst_d.at[1], sem4)
            pltpu.make_async_copy(g_hbm.at[src_at(j)], rows_v, sem).wait()
            pltpu.make_async_copy(dst_hbm_at(j), dst_d.at[0], sem3).wait()
            pltpu.sync_copy(rows_v, acc_sh.at[dst_d.at[0]], add=True)

            @pl.when(jj + 1 < n_pairs)
            def _():
                pltpu.async_copy(g_hbm.at[src_at(j + 2)], rows_v, sem)
                pltpu.async_copy(dst_hbm_at(j + 2), dst_d.at[0], sem3)

            pltpu.make_async_copy(g_hbm.at[src_at(j + 1)], rows_w,
                                  sem2).wait()
            pltpu.make_async_copy(dst_hbm_at(j + 1), dst_d.at[1],
                                  sem4).wait()
            pltpu.sync_copy(rows_w, acc_sh.at[dst_d.at[1]], add=True)
            return 0

        lax.fori_loop(0, n_pairs, pair_body, 0)
        plsc.subcore_barrier()
        pltpu.sync_copy(acc_sh.at[pl.ds(base, rows_per_tile)],
                        out_hbm.at[c, pl.ds(base, rows_per_tile)])

    return agg_kernel


def _dinv_col(degp_block):
    deg_row = jnp.sum(degp_block, axis=0, keepdims=True) + 1.0
    return lax.rsqrt(jnp.transpose(deg_row))


def _h_body(x_ref, w_ref, h_ref):
    h_ref[...] = jnp.dot(x_ref[...], w_ref[...],
                         preferred_element_type=jnp.float32)


def _g_body(h_ref, degp_ref, g_ref):
    g_ref[...] = h_ref[...] * _dinv_col(degp_ref[...])


def _out_body(acc_ref, g_ref, degp_ref, b_ref, o_ref):
    a = acc_ref[0] + acc_ref[1]
    o_ref[...] = _dinv_col(degp_ref[...]) * (a + g_ref[...]) + b_ref[...]


def kernel(x, edge_index, W, b):
    n, d_in = x.shape
    d_out = W.shape[1]
    e = edge_index.shape[1]

    rpt = (((n + 1 + NS - 1) // NS + 7) // 8) * 8
    nacc = rpt * NS
    n_chunks = (e + NW * CHUNK - 1) // (NW * CHUNK)
    n_chunks += n_chunks % 2
    ept = n_chunks * CHUNK
    bw = e // ept
    off_e = e - bw * ept
    assert off_e % (2 * CHUNK) == 0 and e % LANES == 0

    edges = edge_index.astype(jnp.int32)
    degp = _make_deg_kernel(nacc, n_chunks, bw, off_e)(edges)

    bn = 1024
    grid = ((n + bn - 1) // bn,)
    h = pl.pallas_call(
        _h_body,
        grid=grid,
        in_specs=[
            pl.BlockSpec((bn, d_in), lambda i: (i, 0)),
            pl.BlockSpec((d_in, d_out), lambda i: (0, 0)),
        ],
        out_specs=pl.BlockSpec((bn, d_out), lambda i: (i, 0)),
        out_shape=jax.ShapeDtypeStruct((n, d_out), jnp.float32),
    )(x, W)

    g = pl.pallas_call(
        _g_body,
        grid=grid,
        in_specs=[
            pl.BlockSpec((bn, d_out), lambda i: (i, 0)),
            pl.BlockSpec((NW, bn), lambda i: (0, i)),
        ],
        out_specs=pl.BlockSpec((bn, d_out), lambda i: (i, 0)),
        out_shape=jax.ShapeDtypeStruct((n, d_out), jnp.float32),
    )(h, degp)

    acc = _make_agg_kernel(nacc, d_out, n_chunks, bw, off_e)(edges, g)

    out = pl.pallas_call(
        _out_body,
        grid=grid,
        in_specs=[
            pl.BlockSpec((NC, bn, d_out), lambda i: (0, i, 0)),
            pl.BlockSpec((bn, d_out), lambda i: (i, 0)),
            pl.BlockSpec((NW, bn), lambda i: (0, i)),
            pl.BlockSpec((1, d_out), lambda i: (0, 0)),
        ],
        out_specs=pl.BlockSpec((bn, d_out), lambda i: (i, 0)),
        out_shape=jax.ShapeDtypeStruct((n, d_out), jnp.float32),
    )(acc, g, degp, b.reshape(1, d_out))
    return out


# speedup vs baseline: 3.4212x; 1.0267x over previous
import functools

import jax
import jax.numpy as jnp
from jax import lax
from jax.experimental import pallas as pl
from jax.experimental.pallas import tpu as pltpu
from jax.experimental.pallas import tpu_sc as plsc

NC = 2
NS = 16
NW = NC * NS
LANES = 16
CHUNK = 128


def _make_deg_kernel(nacc, n_chunks, bw, off_e):
    mesh = plsc.VectorSubcoreMesh(core_axis_name="c", subcore_axis_name="s")
    ept = n_chunks * CHUNK

    @functools.partial(
        pl.kernel,
        out_type=jax.ShapeDtypeStruct((NW, nacc), jnp.float32),
        mesh=mesh,
        scratch_types=[
            pltpu.VMEM((ept,), jnp.int32),
            pltpu.VMEM((nacc,), jnp.float32),
        ],
        compiler_params=pltpu.CompilerParams(needs_layout_passes=False),
    )
    def deg_kernel(edges_hbm, out_hbm, dst_v, deg_v):
        c = lax.axis_index("c")
        s = lax.axis_index("s")
        wid = s * NC + c
        zeros16 = jnp.zeros((LANES,), jnp.float32)

        def zero_body(i, _):
            deg_v[pl.ds(i * LANES, LANES)] = zeros16
            return 0

        lax.fori_loop(0, nacc // LANES, zero_body, 0)

        @pl.when(wid < bw)
        def _():
            pltpu.sync_copy(edges_hbm.at[1, pl.ds(wid * ept, ept)], dst_v)

        if off_e:
            @pl.when(wid == bw)
            def _():
                pltpu.sync_copy(edges_hbm.at[1, pl.ds(bw * ept, off_e)],
                                dst_v.at[pl.ds(0, off_e)])

        n_real = jnp.where(wid < bw, ept, jnp.where(wid == bw, off_e, 0))
        ones16 = jnp.ones((LANES,), jnp.float32)

        def body(i, _):
            for u in range(4):
                idx = dst_v[pl.ds(i * (4 * LANES) + u * LANES, LANES)]
                plsc.addupdate_scatter(deg_v, [idx], ones16)
            return 0

        lax.fori_loop(0, n_real // (4 * LANES), body, 0)
        pltpu.sync_copy(deg_v, out_hbm.at[wid])

    return deg_kernel


def _make_agg_kernel(nacc, d, n_chunks, bw, off_e):
    mesh = plsc.VectorSubcoreMesh(core_axis_name="c", subcore_axis_name="s")
    rows_per_tile = nacc // NS
    ept = n_chunks * CHUNK

    @functools.partial(
        pl.kernel,
        out_type=jax.ShapeDtypeStruct((NC, nacc, d), jnp.float32),
        mesh=mesh,
        scratch_types=[
            pltpu.VMEM((ept,), jnp.int32),
            pltpu.VMEM((2, CHUNK), jnp.int32),
            pltpu.VMEM((CHUNK, d), jnp.float32),
            pltpu.VMEM((CHUNK, d), jnp.float32),
            pltpu.VMEM_SHARED((nacc, d), jnp.float32),
            pltpu.SemaphoreType.DMA,
            pltpu.SemaphoreType.DMA,
            pltpu.SemaphoreType.DMA,
            pltpu.SemaphoreType.DMA,
        ],
        compiler_params=pltpu.CompilerParams(needs_layout_passes=False),
    )
    def agg_kernel(edges_hbm, g_hbm, out_hbm, src_v, dst_d, rows_v,
                   rows_w, acc_sh, sem, sem2, sem3, sem4):
        c = lax.axis_index("c")
        s = lax.axis_index("s")
        wid = s * NC + c
        ebase = wid * ept
        zeros16 = jnp.zeros((LANES,), jnp.float32)

        @pl.when(wid < bw)
        def _():
            pltpu.async_copy(edges_hbm.at[0, pl.ds(ebase, ept)], src_v, sem2)

        if off_e:
            @pl.when(wid == bw)
            def _():
                pltpu.async_copy(edges_hbm.at[0, pl.ds(bw * ept, off_e)],
                                 src_v.at[pl.ds(0, off_e)], sem2)

        def zero_rows(i, _):
            for j in range(d // LANES):
                rows_v[i, pl.ds(j * LANES, LANES)] = zeros16
            return 0

        lax.fori_loop(0, CHUNK, zero_rows, 0)
        base = s * rows_per_tile

        def zero_acc(k, _):
            pltpu.sync_copy(rows_v, acc_sh.at[pl.ds(base + k * CHUNK, CHUNK)])
            return 0

        lax.fori_loop(0, rows_per_tile // CHUNK, zero_acc, 0)
        rem = rows_per_tile % CHUNK
        if rem:
            pltpu.sync_copy(
                rows_v.at[pl.ds(0, rem)],
                acc_sh.at[pl.ds(base + (rows_per_tile // CHUNK) * CHUNK, rem)],
            )
        plsc.subcore_barrier()

        @pl.when(wid < bw)
        def _():
            pltpu.make_async_copy(edges_hbm.at[0, pl.ds(ebase, ept)], src_v,
                                  sem2).wait()

        if off_e:
            @pl.when(wid == bw)
            def _():
                pltpu.make_async_copy(edges_hbm.at[0, pl.ds(bw * ept, off_e)],
                                      src_v.at[pl.ds(0, off_e)], sem2).wait()

        n_real = jnp.where(wid < bw, ept, jnp.where(wid == bw, off_e, 0))
        n_pairs = n_real // (2 * CHUNK)

        def src_at(j):
            return src_v.at[pl.ds(j * CHUNK, CHUNK)]

        def dst_hbm_at(j):
            return edges_hbm.at[1, pl.ds(ebase + j * CHUNK, CHUNK)]

        @pl.when(n_pairs > 0)
        def _():
            pltpu.async_copy(g_hbm.at[src_at(0)], rows_v, sem)
            pltpu.async_copy(dst_hbm_at(0), dst_d.at[0], sem3)

        def pair_body(jj, _):
            j = 2 * jj
            pltpu.async_copy(g_hbm.at[src_at(j + 1)], rows_w, sem2)
            pltpu.async_copy(dst_hbm_at(j + 1), dst_d.at[1], sem4)
            pltpu.make_async_copy(g_hbm.at[src_at(j)], rows_v, sem).wait()
            pltpu.make_async_copy(dst_hbm_at(j), dst_d.at[0], sem3).wait()
            pltpu.sync_copy(rows_v, acc_sh.at[dst_d.at[0]], add=True)

            @pl.when(jj + 1 < n_pairs)
            def _():
                pltpu.async_copy(g_hbm.at[src_at(j + 2)], rows_v, sem)
                pltpu.async_copy(dst_hbm_at(j + 2), dst_d.at[0], sem3)

            pltpu.make_async_copy(g_hbm.at[src_at(j + 1)], rows_w,
                                  sem2).wait()
            pltpu.make_async_copy(dst_hbm_at(j + 1), dst_d.at[1],
                                  sem4).wait()
            pltpu.sync_copy(rows_w, acc_sh.at[dst_d.at[1]], add=True)
            return 0

        lax.fori_loop(0, n_pairs, pair_body, 0)
        plsc.subcore_barrier()
        pltpu.sync_copy(acc_sh.at[pl.ds(base, rows_per_tile)],
                        out_hbm.at[c, pl.ds(base, rows_per_tile)])

    return agg_kernel


def _dinv_col(degp_block):
    deg_row = jnp.sum(degp_block, axis=0, keepdims=True) + 1.0
    return lax.rsqrt(jnp.transpose(deg_row))


def _h_body(x_ref, w_ref, h_ref):
    h_ref[...] = jnp.dot(x_ref[...], w_ref[...],
                         preferred_element_type=jnp.float32)


def _g_body(h_ref, degp_ref, g_ref):
    g_ref[...] = h_ref[...] * _dinv_col(degp_ref[...])


def _out_body(acc_ref, g_ref, degp_ref, b_ref, o_ref):
    a = acc_ref[0] + acc_ref[1]
    o_ref[...] = _dinv_col(degp_ref[...]) * (a + g_ref[...]) + b_ref[...]


def kernel(x, edge_index, W, b):
    n, d_in = x.shape
    d_out = W.shape[1]
    e = edge_index.shape[1]

    rpt = (((n + 1 + NS - 1) // NS + 7) // 8) * 8
    nacc = rpt * NS
    n_chunks = (e + NW * CHUNK - 1) // (NW * CHUNK)
    n_chunks += n_chunks % 2
    ept = n_chunks * CHUNK
    bw = e // ept
    off_e = e - bw * ept
    assert off_e % (2 * CHUNK) == 0 and e % LANES == 0

    edges = edge_index.astype(jnp.int32)
    degp = _make_deg_kernel(nacc, n_chunks, bw, off_e)(edges)

    bn = 2048
    grid = ((n + bn - 1) // bn,)
    h = pl.pallas_call(
        _h_body,
        grid=grid,
        in_specs=[
            pl.BlockSpec((bn, d_in), lambda i: (i, 0)),
            pl.BlockSpec((d_in, d_out), lambda i: (0, 0)),
        ],
        out_specs=pl.BlockSpec((bn, d_out), lambda i: (i, 0)),
        out_shape=jax.ShapeDtypeStruct((n, d_out), jnp.float32),
    )(x, W)

    g = pl.pallas_call(
        _g_body,
        grid=grid,
        in_specs=[
            pl.BlockSpec((bn, d_out), lambda i: (i, 0)),
            pl.BlockSpec((NW, bn), lambda i: (0, i)),
        ],
        out_specs=pl.BlockSpec((bn, d_out), lambda i: (i, 0)),
        out_shape=jax.ShapeDtypeStruct((n, d_out), jnp.float32),
    )(h, degp)

    acc = _make_agg_kernel(nacc, d_out, n_chunks, bw, off_e)(edges, g)

    out = pl.pallas_call(
        _out_body,
        grid=grid,
        in_specs=[
            pl.BlockSpec((NC, bn, d_out), lambda i: (0, i, 0)),
            pl.BlockSpec((bn, d_out), lambda i: (i, 0)),
            pl.BlockSpec((NW, bn), lambda i: (0, i)),
            pl.BlockSpec((1, d_out), lambda i: (0, 0)),
        ],
        out_specs=pl.BlockSpec((bn, d_out), lambda i: (i, 0)),
        out_shape=jax.ShapeDtypeStruct((n, d_out), jnp.float32),
    )(acc, g, degp, b.reshape(1, d_out))
    return out


# speedup vs baseline: 3.4314x; 1.0030x over previous
import functools

import jax
import jax.numpy as jnp
from jax import lax
from jax.experimental import pallas as pl
from jax.experimental.pallas import tpu as pltpu
from jax.experimental.pallas import tpu_sc as plsc

NC = 2
NS = 16
NW = NC * NS
LANES = 16
CHUNK = 128


def _make_deg_kernel(nacc, n_chunks, bw, off_e):
    mesh = plsc.VectorSubcoreMesh(core_axis_name="c", subcore_axis_name="s")
    ept = n_chunks * CHUNK

    @functools.partial(
        pl.kernel,
        out_type=jax.ShapeDtypeStruct((NW, nacc), jnp.float32),
        mesh=mesh,
        scratch_types=[
            pltpu.VMEM((ept,), jnp.int32),
            pltpu.VMEM((nacc,), jnp.float32),
        ],
        compiler_params=pltpu.CompilerParams(needs_layout_passes=False),
    )
    def deg_kernel(edges_hbm, out_hbm, dst_v, deg_v):
        c = lax.axis_index("c")
        s = lax.axis_index("s")
        wid = s * NC + c
        zeros16 = jnp.zeros((LANES,), jnp.float32)

        def zero_body(i, _):
            deg_v[pl.ds(i * LANES, LANES)] = zeros16
            return 0

        lax.fori_loop(0, nacc // LANES, zero_body, 0)

        @pl.when(wid < bw)
        def _():
            pltpu.sync_copy(edges_hbm.at[1, pl.ds(wid * ept, ept)], dst_v)

        if off_e:
            @pl.when(wid == bw)
            def _():
                pltpu.sync_copy(edges_hbm.at[1, pl.ds(bw * ept, off_e)],
                                dst_v.at[pl.ds(0, off_e)])

        n_real = jnp.where(wid < bw, ept, jnp.where(wid == bw, off_e, 0))
        ones16 = jnp.ones((LANES,), jnp.float32)

        def body(i, _):
            for u in range(4):
                idx = dst_v[pl.ds(i * (4 * LANES) + u * LANES, LANES)]
                plsc.addupdate_scatter(deg_v, [idx], ones16)
            return 0

        lax.fori_loop(0, n_real // (4 * LANES), body, 0)
        pltpu.sync_copy(deg_v, out_hbm.at[wid])

    return deg_kernel


def _make_agg_kernel(nacc, d, n_chunks, bw, off_e):
    mesh = plsc.VectorSubcoreMesh(core_axis_name="c", subcore_axis_name="s")
    rows_per_tile = nacc // NS
    ept = n_chunks * CHUNK

    @functools.partial(
        pl.kernel,
        out_type=jax.ShapeDtypeStruct((NC, nacc, d), jnp.float32),
        mesh=mesh,
        scratch_types=[
            pltpu.VMEM((ept,), jnp.int32),
            pltpu.VMEM((2, CHUNK), jnp.int32),
            pltpu.VMEM((CHUNK, d), jnp.float32),
            pltpu.VMEM((CHUNK, d), jnp.float32),
            pltpu.VMEM_SHARED((nacc, d), jnp.float32),
            pltpu.SemaphoreType.DMA,
            pltpu.SemaphoreType.DMA,
            pltpu.SemaphoreType.DMA,
            pltpu.SemaphoreType.DMA,
        ],
        compiler_params=pltpu.CompilerParams(needs_layout_passes=False),
    )
    def agg_kernel(edges_hbm, g_hbm, out_hbm, src_v, dst_d, rows_v,
                   rows_w, acc_sh, sem, sem2, sem3, sem4):
        c = lax.axis_index("c")
        s = lax.axis_index("s")
        wid = s * NC + c
        ebase = wid * ept
        zeros16 = jnp.zeros((LANES,), jnp.float32)

        @pl.when(wid < bw)
        def _():
            pltpu.async_copy(edges_hbm.at[0, pl.ds(ebase, ept)], src_v, sem2)

        if off_e:
            @pl.when(wid == bw)
            def _():
                pltpu.async_copy(edges_hbm.at[0, pl.ds(bw * ept, off_e)],
                                 src_v.at[pl.ds(0, off_e)], sem2)

        def zero_rows(i, _):
            for j in range(d // LANES):
                rows_v[i, pl.ds(j * LANES, LANES)] = zeros16
            return 0

        lax.fori_loop(0, CHUNK, zero_rows, 0)
        base = s * rows_per_tile

        def zero_acc(k, _):
            pltpu.async_copy(rows_v, acc_sh.at[pl.ds(base + k * CHUNK, CHUNK)],
                             sem)
            return 0

        lax.fori_loop(0, rows_per_tile // CHUNK, zero_acc, 0)
        rem = rows_per_tile % CHUNK
        if rem:
            pltpu.async_copy(
                rows_v.at[pl.ds(0, rem)],
                acc_sh.at[pl.ds(base + (rows_per_tile // CHUNK) * CHUNK, rem)],
                sem,
            )

        def zero_wait(k, _):
            pltpu.make_async_copy(
                rows_v, acc_sh.at[pl.ds(base + k * CHUNK, CHUNK)], sem).wait()
            return 0

        lax.fori_loop(0, rows_per_tile // CHUNK, zero_wait, 0)
        if rem:
            pltpu.make_async_copy(
                rows_v.at[pl.ds(0, rem)],
                acc_sh.at[pl.ds(base + (rows_per_tile // CHUNK) * CHUNK, rem)],
                sem,
            ).wait()
        plsc.subcore_barrier()

        @pl.when(wid < bw)
        def _():
            pltpu.make_async_copy(edges_hbm.at[0, pl.ds(ebase, ept)], src_v,
                                  sem2).wait()

        if off_e:
            @pl.when(wid == bw)
            def _():
                pltpu.make_async_copy(edges_hbm.at[0, pl.ds(bw * ept, off_e)],
                                      src_v.at[pl.ds(0, off_e)], sem2).wait()

        n_real = jnp.where(wid < bw, ept, jnp.where(wid == bw, off_e, 0))
        n_pairs = n_real // (2 * CHUNK)

        def src_at(j):
            return src_v.at[pl.ds(j * CHUNK, CHUNK)]

        def dst_hbm_at(j):
            return edges_hbm.at[1, pl.ds(ebase + j * CHUNK, CHUNK)]

        @pl.when(n_pairs > 0)
        def _():
            pltpu.async_copy(g_hbm.at[src_at(0)], rows_v, sem)
            pltpu.async_copy(dst_hbm_at(0), dst_d.at[0], sem3)

        def pair_body(jj, _):
            j = 2 * jj
            pltpu.async_copy(g_hbm.at[src_at(j + 1)], rows_w, sem2)
            pltpu.async_copy(dst_hbm_at(j + 1), dst_d.at[1], sem4)
            pltpu.make_async_copy(g_hbm.at[src_at(j)], rows_v, sem).wait()
            pltpu.make_async_copy(dst_hbm_at(j), dst_d.at[0], sem3).wait()
            pltpu.sync_copy(rows_v, acc_sh.at[dst_d.at[0]], add=True)

            @pl.when(jj + 1 < n_pairs)
            def _():
                pltpu.async_copy(g_hbm.at[src_at(j + 2)], rows_v, sem)
                pltpu.async_copy(dst_hbm_at(j + 2), dst_d.at[0], sem3)

            pltpu.make_async_copy(g_hbm.at[src_at(j + 1)], rows_w,
                                  sem2).wait()
            pltpu.make_async_copy(dst_hbm_at(j + 1), dst_d.at[1],
                                  sem4).wait()
            pltpu.sync_copy(rows_w, acc_sh.at[dst_d.at[1]], add=True)
            return 0

        lax.fori_loop(0, n_pairs, pair_body, 0)
        plsc.subcore_barrier()
        pltpu.sync_copy(acc_sh.at[pl.ds(base, rows_per_tile)],
                        out_hbm.at[c, pl.ds(base, rows_per_tile)])

    return agg_kernel


def _dinv_col(degp_block):
    deg_row = jnp.sum(degp_block, axis=0, keepdims=True) + 1.0
    return lax.rsqrt(jnp.transpose(deg_row))


def _h_body(x_ref, w_ref, h_ref):
    h_ref[...] = jnp.dot(x_ref[...], w_ref[...],
                         preferred_element_type=jnp.float32)


def _g_body(h_ref, degp_ref, g_ref):
    g_ref[...] = h_ref[...] * _dinv_col(degp_ref[...])


def _out_body(acc_ref, g_ref, degp_ref, b_ref, o_ref):
    a = acc_ref[0] + acc_ref[1]
    o_ref[...] = _dinv_col(degp_ref[...]) * (a + g_ref[...]) + b_ref[...]


def kernel(x, edge_index, W, b):
    n, d_in = x.shape
    d_out = W.shape[1]
    e = edge_index.shape[1]

    rpt = (((n + 1 + NS - 1) // NS + 7) // 8) * 8
    nacc = rpt * NS
    n_chunks = (e + NW * CHUNK - 1) // (NW * CHUNK)
    n_chunks += n_chunks % 2
    ept = n_chunks * CHUNK
    bw = e // ept
    off_e = e - bw * ept
    assert off_e % (2 * CHUNK) == 0 and e % LANES == 0

    edges = edge_index.astype(jnp.int32)
    degp = _make_deg_kernel(nacc, n_chunks, bw, off_e)(edges)

    bn = 2048
    grid = ((n + bn - 1) // bn,)
    h = pl.pallas_call(
        _h_body,
        grid=grid,
        in_specs=[
            pl.BlockSpec((bn, d_in), lambda i: (i, 0)),
            pl.BlockSpec((d_in, d_out), lambda i: (0, 0)),
        ],
        out_specs=pl.BlockSpec((bn, d_out), lambda i: (i, 0)),
        out_shape=jax.ShapeDtypeStruct((n, d_out), jnp.float32),
    )(x, W)

    g = pl.pallas_call(
        _g_body,
        grid=grid,
        in_specs=[
            pl.BlockSpec((bn, d_out), lambda i: (i, 0)),
            pl.BlockSpec((NW, bn), lambda i: (0, i)),
        ],
        out_specs=pl.BlockSpec((bn, d_out), lambda i: (i, 0)),
        out_shape=jax.ShapeDtypeStruct((n, d_out), jnp.float32),
    )(h, degp)

    acc = _make_agg_kernel(nacc, d_out, n_chunks, bw, off_e)(edges, g)

    out = pl.pallas_call(
        _out_body,
        grid=grid,
        in_specs=[
            pl.BlockSpec((NC, bn, d_out), lambda i: (0, i, 0)),
            pl.BlockSpec((bn, d_out), lambda i: (i, 0)),
            pl.BlockSpec((NW, bn), lambda i: (0, i)),
            pl.BlockSpec((1, d_out), lambda i: (0, 0)),
        ],
        out_specs=pl.BlockSpec((bn, d_out), lambda i: (i, 0)),
        out_shape=jax.ShapeDtypeStruct((n, d_out), jnp.float32),
    )(acc, g, degp, b.reshape(1, d_out))
    return out
